# Initial kernel scaffold; baseline (speedup 1.0000x reference)
import jax
import jax.numpy as jnp
from jax.experimental import pallas as pl


def kernel(x, edge_index, Wp, bp, Wl1, bl1, Wr1, Wl2, bl2, Wr2):
    raise NotImplementedError("write your pallas kernel here")


# speedup vs baseline: 4.4792x; 4.4792x over previous
import functools

import jax
import jax.numpy as jnp
from jax import lax
from jax.experimental import pallas as pl
from jax.experimental.pallas import tpu as pltpu
from jax.experimental.pallas import tpu_sc as plsc

N_NODES = 10000
N_EDGES = 320000
D = 128

NC = 2
NS = 16
NW = NC * NS

EDGE_BATCH = 128
E_PAD = 2560 * EDGE_BATCH
ROWS_PER_W = E_PAD // EDGE_BATCH // NW
ACC_ROWS = 10240
TRASH = N_NODES
SUB_ROWS = ACC_ROWS // NS


K_IDX = 8
DEG_W = 8


def _make_sc_agg(ncols, g_rows, with_deg):
  n_outer = ROWS_PER_W // K_IDX
  mesh = plsc.VectorSubcoreMesh(core_axis_name="c", subcore_axis_name="s",
                                num_cores=NC, num_subcores=NS)

  out_type = [jax.ShapeDtypeStruct((NC, ACC_ROWS, ncols), jnp.float32)]
  scratch = [
      pltpu.MemorySpace.VMEM_SHARED((ACC_ROWS, ncols), jnp.float32),
      pltpu.VMEM((K_IDX, EDGE_BATCH), jnp.int32),
      pltpu.VMEM((K_IDX, EDGE_BATCH), jnp.int32),
      pltpu.VMEM((g_rows, EDGE_BATCH, ncols), jnp.float32),
      pltpu.SemaphoreType.DMA,
  ]
  if with_deg:
    out_type.append(jax.ShapeDtypeStruct((NC, ACC_ROWS, DEG_W), jnp.float32))
    scratch += [
        pltpu.MemorySpace.VMEM_SHARED((ACC_ROWS, DEG_W), jnp.float32),
        pltpu.VMEM((EDGE_BATCH, DEG_W), jnp.float32),
    ]

  def body(table, src2d, dst2d, zeros_hbm, ones_hbm, zdeg_hbm,
           out, deg_out, acc, idx_s, idx_d, rows, sem, deg_acc=None,
           ones=None):
    c = lax.axis_index("c")
    s = lax.axis_index("s")
    wid = c * NS + s

    pltpu.sync_copy(zeros_hbm.at[pl.ds(s * SUB_ROWS, SUB_ROWS)],
                    acc.at[pl.ds(s * SUB_ROWS, SUB_ROWS)])
    if with_deg:
      pltpu.sync_copy(zdeg_hbm.at[pl.ds(s * SUB_ROWS, SUB_ROWS)],
                      deg_acc.at[pl.ds(s * SUB_ROWS, SUB_ROWS)])
      pltpu.sync_copy(ones_hbm, ones)
    plsc.subcore_barrier()

    @pl.loop(0, n_outer)
    def _(it):
      base = wid * ROWS_PER_W + it * K_IDX
      pltpu.sync_copy(src2d.at[pl.ds(base, K_IDX)], idx_s)
      pltpu.sync_copy(dst2d.at[pl.ds(base, K_IDX)], idx_d)
      for g0 in range(0, K_IDX, g_rows):
        copies = [
            pltpu.async_copy(table.at[idx_s.at[g0 + j]], rows.at[j], sem)
            for j in range(g_rows)
        ]
        for cp in copies:
          cp.wait()
        for j in range(g_rows):
          pltpu.sync_copy(rows.at[j], acc.at[idx_d.at[g0 + j]], add=True)
          if with_deg:
            pltpu.sync_copy(ones, deg_acc.at[idx_d.at[g0 + j]], add=True)

    plsc.subcore_barrier()
    pltpu.sync_copy(acc.at[pl.ds(s * SUB_ROWS, SUB_ROWS)],
                    out.at[c, pl.ds(s * SUB_ROWS, SUB_ROWS)])
    if with_deg:
      pltpu.sync_copy(deg_acc.at[pl.ds(s * SUB_ROWS, SUB_ROWS)],
                      deg_out.at[c, pl.ds(s * SUB_ROWS, SUB_ROWS)])

  if with_deg:
    def body_wd(table, src2d, dst2d, z, o, zd, out, deg_out,
                acc, idx_s, idx_d, rows, sem, deg_acc, ones):
      body(table, src2d, dst2d, z, o, zd, out, deg_out, acc, idx_s, idx_d,
           rows, sem, deg_acc, ones)
    fn = body_wd
  else:
    def body_nd(table, src2d, dst2d, z, o, zd, out,
                acc, idx_s, idx_d, rows, sem):
      body(table, src2d, dst2d, z, o, zd, out, None, acc, idx_s, idx_d,
           rows, sem)
    fn = body_nd

  return pl.kernel(fn, out_type=tuple(out_type), mesh=mesh,
                   scratch_types=scratch,
                   compiler_params=pltpu.CompilerParams(
                       use_tc_tiling_on_sc=False))


def _tc_pre(x, WpT, bp):
  def body(x_ref, w_ref, b_ref, o_ref):
    o_ref[...] = jnp.maximum(
        jnp.dot(x_ref[...], w_ref[...],
                preferred_element_type=jnp.float32) + b_ref[...], 0.0)
  r = 2000
  return pl.pallas_call(
      body,
      grid=(N_NODES // r,),
      in_specs=[
          pl.BlockSpec((r, D), lambda i: (i, 0)),
          pl.BlockSpec((D, D), lambda i: (0, 0)),
          pl.BlockSpec((1, D), lambda i: (0, 0)),
      ],
      out_specs=pl.BlockSpec((r, D), lambda i: (i, 0)),
      out_shape=jax.ShapeDtypeStruct((N_NODES, D), jnp.float32),
  )(x, WpT, bp)


def _tc_mid(agg0, agg1, deg0, deg1, h, Wl1T, bl1, Wr1T, Wl2Tp, Wr2Tp):
  def body(a0, a1, d0, d1, h_ref, wl, bl, wr, w2l, w2r, g1_ref, r2_ref):
    deg = d0[...] + d1[...]
    recip = 1.0 / jnp.maximum(deg[:, 0:1], 1.0)
    mean = (a0[...] + a1[...]) * recip
    h1 = jnp.maximum(
        jnp.dot(mean, wl[...], preferred_element_type=jnp.float32)
        + bl[...]
        + jnp.dot(h_ref[...], wr[...], preferred_element_type=jnp.float32),
        0.0)
    g1_ref[...] = jnp.dot(h1, w2l[...], preferred_element_type=jnp.float32)
    r2_ref[...] = jnp.dot(h1, w2r[...], preferred_element_type=jnp.float32)
  r = 2000
  full = lambda i: (0, 0)
  return pl.pallas_call(
      body,
      grid=(N_NODES // r,),
      in_specs=[
          pl.BlockSpec((r, D), lambda i: (i, 0)),
          pl.BlockSpec((r, D), lambda i: (i, 0)),
          pl.BlockSpec((r, DEG_W), lambda i: (i, 0)),
          pl.BlockSpec((r, DEG_W), lambda i: (i, 0)),
          pl.BlockSpec((r, D), lambda i: (i, 0)),
          pl.BlockSpec((D, D), full),
          pl.BlockSpec((1, D), full),
          pl.BlockSpec((D, D), full),
          pl.BlockSpec((D, 16), full),
          pl.BlockSpec((D, 16), full),
      ],
      out_specs=[
          pl.BlockSpec((r, 16), lambda i: (i, 0)),
          pl.BlockSpec((r, 16), lambda i: (i, 0)),
      ],
      out_shape=[
          jax.ShapeDtypeStruct((N_NODES, 16), jnp.float32),
          jax.ShapeDtypeStruct((N_NODES, 16), jnp.float32),
      ],
  )(agg0, agg1, deg0, deg1, h, Wl1T, bl1, Wr1T, Wl2Tp, Wr2Tp)


def _tc_final(agg2_0, agg2_1, deg0, deg1, r2, bl2p):
  def body(a0, a1, d0, d1, r2_ref, b_ref, o_ref):
    deg = d0[...] + d1[...]
    recip = 1.0 / jnp.maximum(deg[:, 0:1], 1.0)
    o = (a0[...] + a1[...]) * recip + r2_ref[...] + b_ref[...]
    o_ref[...] = o[:, 0:4]
  r = 2000
  return pl.pallas_call(
      body,
      grid=(N_NODES // r,),
      in_specs=[
          pl.BlockSpec((r, 16), lambda i: (i, 0)),
          pl.BlockSpec((r, 16), lambda i: (i, 0)),
          pl.BlockSpec((r, DEG_W), lambda i: (i, 0)),
          pl.BlockSpec((r, DEG_W), lambda i: (i, 0)),
          pl.BlockSpec((r, 16), lambda i: (i, 0)),
          pl.BlockSpec((1, 16), lambda i: (0, 0)),
      ],
      out_specs=pl.BlockSpec((r, 4), lambda i: (i, 0)),
      out_shape=jax.ShapeDtypeStruct((N_NODES, 4), jnp.float32),
  )(agg2_0, agg2_1, deg0, deg1, r2, bl2p)


_sc_agg_wide = functools.lru_cache(None)(
    lambda: _make_sc_agg(D, 1, with_deg=True))
_sc_agg_narrow = functools.lru_cache(None)(
    lambda: _make_sc_agg(16, 8, with_deg=False))


def kernel(x, edge_index, Wp, bp, Wl1, bl1, Wr1, Wl2, bl2, Wr2):
  src = edge_index[0].astype(jnp.int32)
  dst = edge_index[1].astype(jnp.int32)
  npad = E_PAD - N_EDGES
  src2d = jnp.concatenate(
      [src, jnp.zeros((npad,), jnp.int32)]).reshape(-1, EDGE_BATCH)
  dst2d = jnp.concatenate(
      [dst, jnp.full((npad,), TRASH, jnp.int32)]).reshape(-1, EDGE_BATCH)

  zeros_w = jnp.zeros((ACC_ROWS, D), jnp.float32)
  zeros_16 = jnp.zeros((ACC_ROWS, 16), jnp.float32)
  zeros_d = jnp.zeros((ACC_ROWS, DEG_W), jnp.float32)
  ones = jnp.ones((EDGE_BATCH, DEG_W), jnp.float32)

  h = _tc_pre(x, Wp.T, bp.reshape(1, D))

  agg, deg = _sc_agg_wide()(h, src2d, dst2d, zeros_w, ones, zeros_d)

  Wl2Tp = jnp.zeros((D, 16), jnp.float32).at[:, 0:4].set(Wl2.T)
  Wr2Tp = jnp.zeros((D, 16), jnp.float32).at[:, 0:4].set(Wr2.T)
  g1, r2 = _tc_mid(agg[0], agg[1], deg[0], deg[1], h,
                   Wl1.T, bl1.reshape(1, D), Wr1.T, Wl2Tp, Wr2Tp)

  (agg2,) = _sc_agg_narrow()(g1, src2d, dst2d, zeros_16, ones, zeros_16)

  bl2p = jnp.zeros((1, 16), jnp.float32).at[0, 0:4].set(bl2)
  return _tc_final(agg2[0], agg2[1], deg[0], deg[1], r2, bl2p)


# speedup vs baseline: 4.9027x; 1.0945x over previous
import functools

import jax
import jax.numpy as jnp
from jax import lax
from jax.experimental import pallas as pl
from jax.experimental.pallas import tpu as pltpu
from jax.experimental.pallas import tpu_sc as plsc

N_NODES = 10000
N_EDGES = 320000
D = 128
DH = 64

NC = 2
NS = 16
NW = NC * NS

EDGE_BATCH = 128
E_PAD = 2560 * EDGE_BATCH
ROWS_PER_W = E_PAD // EDGE_BATCH // NW
ACC_ROWS = 10240
TRASH = N_NODES
SUB_ROWS = ACC_ROWS // NS

K_IDX = 8
DEG_W = 8
NBUF = 4
PF = 3


def _make_sc_agg(ncols, with_deg):
  n_outer = ROWS_PER_W // K_IDX
  mesh = plsc.VectorSubcoreMesh(core_axis_name="c", subcore_axis_name="s",
                                num_cores=NC, num_subcores=NS)

  out_type = [jax.ShapeDtypeStruct((NC, ACC_ROWS, ncols), jnp.float32)]
  scratch = [
      pltpu.MemorySpace.VMEM_SHARED((ACC_ROWS, ncols), jnp.float32),
      pltpu.VMEM((K_IDX, EDGE_BATCH), jnp.int32),
      pltpu.VMEM((K_IDX, EDGE_BATCH), jnp.int32),
      pltpu.VMEM((NBUF, EDGE_BATCH, ncols), jnp.float32),
  ] + [pltpu.SemaphoreType.DMA] * NBUF
  if with_deg:
    out_type.append(jax.ShapeDtypeStruct((NC, ACC_ROWS, DEG_W), jnp.float32))
    scratch += [
        pltpu.MemorySpace.VMEM_SHARED((ACC_ROWS, DEG_W), jnp.float32),
        pltpu.VMEM((EDGE_BATCH, DEG_W), jnp.float32),
    ]

  def body(table, src2d, dst2d, zeros_hbm, ones_hbm, zdeg_hbm,
           out, deg_out, acc, idx_s, idx_d, rows, sems, deg_acc, ones):
    c = lax.axis_index("c")
    s = lax.axis_index("s")
    wid = c * NS + s

    pltpu.sync_copy(zeros_hbm.at[pl.ds(s * SUB_ROWS, SUB_ROWS)],
                    acc.at[pl.ds(s * SUB_ROWS, SUB_ROWS)])
    if with_deg:
      pltpu.sync_copy(zdeg_hbm.at[pl.ds(s * SUB_ROWS, SUB_ROWS)],
                      deg_acc.at[pl.ds(s * SUB_ROWS, SUB_ROWS)])
      pltpu.sync_copy(ones_hbm, ones)
    plsc.subcore_barrier()

    @pl.loop(0, n_outer)
    def _(it):
      base = wid * ROWS_PER_W + it * K_IDX
      pltpu.sync_copy(src2d.at[pl.ds(base, K_IDX)], idx_s)
      pltpu.sync_copy(dst2d.at[pl.ds(base, K_IDX)], idx_d)
      descs = [
          pltpu.async_copy(table.at[idx_s.at[j]], rows.at[j % NBUF],
                           sems[j % NBUF])
          for j in range(PF)
      ]
      for j in range(K_IDX):
        n = j + PF
        if n < K_IDX:
          descs.append(
              pltpu.async_copy(table.at[idx_s.at[n]], rows.at[n % NBUF],
                               sems[n % NBUF]))
        descs[j].wait()
        pltpu.sync_copy(rows.at[j % NBUF], acc.at[idx_d.at[j]], add=True)
        if with_deg:
          pltpu.sync_copy(ones, deg_acc.at[idx_d.at[j]], add=True)

    plsc.subcore_barrier()
    pltpu.sync_copy(acc.at[pl.ds(s * SUB_ROWS, SUB_ROWS)],
                    out.at[c, pl.ds(s * SUB_ROWS, SUB_ROWS)])
    if with_deg:
      pltpu.sync_copy(deg_acc.at[pl.ds(s * SUB_ROWS, SUB_ROWS)],
                      deg_out.at[c, pl.ds(s * SUB_ROWS, SUB_ROWS)])

  if with_deg:
    def body_wd(table, src2d, dst2d, z, o, zd, out, deg_out,
                acc, idx_s, idx_d, rows, s0, s1, s2, s3, deg_acc, ones):
      body(table, src2d, dst2d, z, o, zd, out, deg_out, acc, idx_s, idx_d,
           rows, (s0, s1, s2, s3), deg_acc, ones)
    fn = body_wd
  else:
    def body_nd(table, src2d, dst2d, z, o, zd, out,
                acc, idx_s, idx_d, rows, s0, s1, s2, s3):
      body(table, src2d, dst2d, z, o, zd, out, None, acc, idx_s, idx_d,
           rows, (s0, s1, s2, s3), None, None)
    fn = body_nd

  return pl.kernel(fn, out_type=tuple(out_type), mesh=mesh,
                   scratch_types=scratch,
                   compiler_params=pltpu.CompilerParams(
                       use_tc_tiling_on_sc=False))


def _tc_pre(x, WpT, bp):
  def body(x_ref, w_ref, b_ref, lo_ref, hi_ref):
    h = jnp.maximum(
        jnp.dot(x_ref[...], w_ref[...],
                preferred_element_type=jnp.float32) + b_ref[...], 0.0)
    lo_ref[...] = h[:, :DH]
    hi_ref[...] = h[:, DH:]
  r = 2000
  return pl.pallas_call(
      body,
      grid=(N_NODES // r,),
      in_specs=[
          pl.BlockSpec((r, D), lambda i: (i, 0)),
          pl.BlockSpec((D, D), lambda i: (0, 0)),
          pl.BlockSpec((1, D), lambda i: (0, 0)),
      ],
      out_specs=[
          pl.BlockSpec((r, DH), lambda i: (i, 0)),
          pl.BlockSpec((r, DH), lambda i: (i, 0)),
      ],
      out_shape=[
          jax.ShapeDtypeStruct((N_NODES, DH), jnp.float32),
          jax.ShapeDtypeStruct((N_NODES, DH), jnp.float32),
      ],
  )(x, WpT, bp)


def _tc_mid(alo0, alo1, ahi0, ahi1, deg0, deg1, h_lo, h_hi,
            Wl1T, bl1, Wr1T, Wl2Tp, Wr2Tp):
  def body(al0, al1, ah0, ah1, d0, d1, hl, hh, wl, bl, wr, w2l, w2r,
           g1_ref, r2_ref):
    deg = d0[...] + d1[...]
    recip = 1.0 / jnp.maximum(deg[:, 0:1], 1.0)
    mean = jnp.concatenate(
        [al0[...] + al1[...], ah0[...] + ah1[...]], axis=1) * recip
    h = jnp.concatenate([hl[...], hh[...]], axis=1)
    h1 = jnp.maximum(
        jnp.dot(mean, wl[...], preferred_element_type=jnp.float32)
        + bl[...]
        + jnp.dot(h, wr[...], preferred_element_type=jnp.float32),
        0.0)
    g1_ref[...] = jnp.dot(h1, w2l[...], preferred_element_type=jnp.float32)
    r2_ref[...] = jnp.dot(h1, w2r[...], preferred_element_type=jnp.float32)
  r = 2000
  full = lambda i: (0, 0)
  return pl.pallas_call(
      body,
      grid=(N_NODES // r,),
      in_specs=[
          pl.BlockSpec((r, DH), lambda i: (i, 0)),
          pl.BlockSpec((r, DH), lambda i: (i, 0)),
          pl.BlockSpec((r, DH), lambda i: (i, 0)),
          pl.BlockSpec((r, DH), lambda i: (i, 0)),
          pl.BlockSpec((r, DEG_W), lambda i: (i, 0)),
          pl.BlockSpec((r, DEG_W), lambda i: (i, 0)),
          pl.BlockSpec((r, DH), lambda i: (i, 0)),
          pl.BlockSpec((r, DH), lambda i: (i, 0)),
          pl.BlockSpec((D, D), full),
          pl.BlockSpec((1, D), full),
          pl.BlockSpec((D, D), full),
          pl.BlockSpec((D, 16), full),
          pl.BlockSpec((D, 16), full),
      ],
      out_specs=[
          pl.BlockSpec((r, 16), lambda i: (i, 0)),
          pl.BlockSpec((r, 16), lambda i: (i, 0)),
      ],
      out_shape=[
          jax.ShapeDtypeStruct((N_NODES, 16), jnp.float32),
          jax.ShapeDtypeStruct((N_NODES, 16), jnp.float32),
      ],
  )(alo0, alo1, ahi0, ahi1, deg0, deg1, h_lo, h_hi,
    Wl1T, bl1, Wr1T, Wl2Tp, Wr2Tp)


def _tc_final(agg2_0, agg2_1, deg0, deg1, r2, bl2p):
  def body(a0, a1, d0, d1, r2_ref, b_ref, o_ref):
    deg = d0[...] + d1[...]
    recip = 1.0 / jnp.maximum(deg[:, 0:1], 1.0)
    o = (a0[...] + a1[...]) * recip + r2_ref[...] + b_ref[...]
    o_ref[...] = o[:, 0:4]
  r = 2000
  return pl.pallas_call(
      body,
      grid=(N_NODES // r,),
      in_specs=[
          pl.BlockSpec((r, 16), lambda i: (i, 0)),
          pl.BlockSpec((r, 16), lambda i: (i, 0)),
          pl.BlockSpec((r, DEG_W), lambda i: (i, 0)),
          pl.BlockSpec((r, DEG_W), lambda i: (i, 0)),
          pl.BlockSpec((r, 16), lambda i: (i, 0)),
          pl.BlockSpec((1, 16), lambda i: (0, 0)),
      ],
      out_specs=pl.BlockSpec((r, 4), lambda i: (i, 0)),
      out_shape=jax.ShapeDtypeStruct((N_NODES, 4), jnp.float32),
  )(agg2_0, agg2_1, deg0, deg1, r2, bl2p)


_sc_half_deg = functools.lru_cache(None)(
    lambda: _make_sc_agg(DH, with_deg=True))
_sc_half = functools.lru_cache(None)(
    lambda: _make_sc_agg(DH, with_deg=False))
_sc_narrow = functools.lru_cache(None)(
    lambda: _make_sc_agg(16, with_deg=False))


def kernel(x, edge_index, Wp, bp, Wl1, bl1, Wr1, Wl2, bl2, Wr2):
  src = edge_index[0].astype(jnp.int32)
  dst = edge_index[1].astype(jnp.int32)
  npad = E_PAD - N_EDGES
  src2d = jnp.concatenate(
      [src, jnp.zeros((npad,), jnp.int32)]).reshape(-1, EDGE_BATCH)
  dst2d = jnp.concatenate(
      [dst, jnp.full((npad,), TRASH, jnp.int32)]).reshape(-1, EDGE_BATCH)

  zeros_h = jnp.zeros((ACC_ROWS, DH), jnp.float32)
  zeros_16 = jnp.zeros((ACC_ROWS, 16), jnp.float32)
  zeros_d = jnp.zeros((ACC_ROWS, DEG_W), jnp.float32)
  ones = jnp.ones((EDGE_BATCH, DEG_W), jnp.float32)

  h_lo, h_hi = _tc_pre(x, Wp.T, bp.reshape(1, D))

  agg_lo, deg = _sc_half_deg()(h_lo, src2d, dst2d, zeros_h, ones, zeros_d)
  (agg_hi,) = _sc_half()(h_hi, src2d, dst2d, zeros_h, ones, zeros_d)

  Wl2Tp = jnp.zeros((D, 16), jnp.float32).at[:, 0:4].set(Wl2.T)
  Wr2Tp = jnp.zeros((D, 16), jnp.float32).at[:, 0:4].set(Wr2.T)
  g1, r2 = _tc_mid(agg_lo[0], agg_lo[1], agg_hi[0], agg_hi[1],
                   deg[0], deg[1], h_lo, h_hi,
                   Wl1.T, bl1.reshape(1, D), Wr1.T, Wl2Tp, Wr2Tp)

  (agg2,) = _sc_narrow()(g1, src2d, dst2d, zeros_16, ones, zeros_16)

  bl2p = jnp.zeros((1, 16), jnp.float32).at[0, 0:4].set(bl2)
  return _tc_final(agg2[0], agg2[1], deg[0], deg[1], r2, bl2p)
